# baseline (speedup 1.0000x reference)
import functools

import jax
import jax.numpy as jnp
from jax import lax
from jax.experimental import pallas as pl
from jax.experimental.pallas import tpu as pltpu
from jax.experimental.pallas import tpu_sc as plsc

NUM_SELECTS = 8
BT = 1024
NC = 2
NS = 16
NW = NC * NS
L = 16


def _mm_body(x_ref, wt_ref, out_ref):
    x_bf = x_ref[...].astype(jnp.bfloat16)
    out_ref[...] = jnp.dot(x_bf, wt_ref[...], preferred_element_type=jnp.float32)


def _matmul(x, wt):
    t, d = x.shape
    e = wt.shape[1]
    return pl.pallas_call(
        _mm_body,
        grid=(t // BT,),
        in_specs=[
            pl.BlockSpec((BT, d), lambda i: (i, 0)),
            pl.BlockSpec((d, e), lambda i: (0, 0)),
        ],
        out_specs=pl.BlockSpec((BT, e), lambda i: (i, 0)),
        out_shape=jax.ShapeDtypeStruct((t, e), jnp.float32),
        compiler_params=pltpu.CompilerParams(
            dimension_semantics=("arbitrary",),
        ),
    )(x, wt)


def _sc_topk(logits):
    t, e = logits.shape
    tw = t // NW
    mesh = plsc.VectorSubcoreMesh(
        core_axis_name="c", subcore_axis_name="s",
        num_cores=NC, num_subcores=NS)

    @functools.partial(
        pl.kernel,
        out_type=[
            jax.ShapeDtypeStruct((t * L,), jnp.int32),
            jax.ShapeDtypeStruct((t * L,), jnp.float32),
        ],
        mesh=mesh,
        scratch_types=[
            pltpu.VMEM((tw, e), jnp.float32),
            pltpu.VMEM((tw * L,), jnp.int32),
            pltpu.VMEM((tw * L,), jnp.float32),
        ],
        compiler_params=pltpu.CompilerParams(needs_layout_passes=False),
    )
    def k(logits_hbm, idx_hbm, sc_hbm, lbuf, ibuf, sbuf):
        wid = lax.axis_index("s") * NC + lax.axis_index("c")
        base = wid * tw
        pltpu.sync_copy(logits_hbm.at[pl.ds(base, tw)], lbuf)
        lane = lax.iota(jnp.int32, L)
        lo8 = lane < NUM_SELECTS

        def body(tok, carry):
            ks = []
            vs = []
            for q in range(e // L):
                key = lbuf[tok, pl.ds(q * L, L)]
                idxv = lane + (q * L)
                sk, sv = plsc.sort_key_val(key, idxv, descending=(q % 2 == 0))
                ks.append(sk)
                vs.append(sv)
            c01k = jnp.where(lo8, ks[0], ks[1])
            c01v = jnp.where(lo8, vs[0], vs[1])
            c23k = jnp.where(lo8, ks[2], ks[3])
            c23v = jnp.where(lo8, vs[2], vs[3])
            d01k, d01v = plsc.sort_key_val(c01k, c01v, descending=True)
            a23k, a23v = plsc.sort_key_val(c23k, c23v, descending=False)
            cfk = jnp.where(lo8, d01k, a23k)
            cfv = jnp.where(lo8, d01v, a23v)
            fk, fv = plsc.sort_key_val(cfk, cfv, descending=True)
            m = jnp.max(fk)
            ex = jnp.where(lo8, jnp.exp(fk - m), jnp.float32(0.0))
            total = jnp.zeros((L,), jnp.float32) + jnp.sum(ex)
            s = ex / total
            ibuf[pl.ds(tok * L, L)] = fv
            sbuf[pl.ds(tok * L, L)] = s
            return carry

        lax.fori_loop(0, tw, body, 0)
        pltpu.sync_copy(ibuf, idx_hbm.at[pl.ds(base * L, tw * L)])
        pltpu.sync_copy(sbuf, sc_hbm.at[pl.ds(base * L, tw * L)])

    return k(logits)


@jax.jit
def kernel(x, gate_weight):
    t, d = x.shape
    wt = gate_weight.T.astype(jnp.bfloat16)
    logits = _matmul(x, wt)
    idx_p, sc_p = _sc_topk(logits)
    idx = idx_p.reshape(t, L)[:, :NUM_SELECTS]
    score = sc_p.reshape(t, L)[:, :NUM_SELECTS]
    return idx, score
